# baseline (speedup 1.0000x reference)
import functools

import jax
import jax.numpy as jnp
from jax import lax
from jax.experimental import pallas as pl
from jax.experimental.pallas import tpu as pltpu
from jax.experimental.pallas import tpu_sc as plsc

_N = 10000
_D = 128
_HD = 32
_E = 320000
_G = 8

_NC = 2
_NSC = 16
_NW = _NC * _NSC
_CB = 128
_NCHUNK = 2560
_EP = _NCHUNK * _CB
_CH = _NCHUNK
_CPW = _CH // _NW
_CPP = _CB // 4
_EPPH = _CH * _CPP
_NP = 10016
_NACC = 10240
_RPT = _NACC // _NSC

_sc_mesh = plsc.VectorSubcoreMesh(
    core_axis_name="c", subcore_axis_name="s", num_cores=_NC, num_subcores=_NSC)


def _k1_body(x_ref, g_ref, b_ref, wa_ref, wb_ref, a_ref, bb_ref):
    x = x_ref[...]
    mu = jnp.mean(x, axis=0, keepdims=True)
    xc = x - mu
    var = jnp.mean(xc * xc, axis=0, keepdims=True)
    xn = xc * (g_ref[...] * lax.rsqrt(var + 1e-5)) + b_ref[...]
    a_ref[pl.ds(0, _N), :] = jnp.dot(xn, wa_ref[...],
                                     preferred_element_type=jnp.float32)
    bb_ref[pl.ds(0, _N), :] = jnp.dot(xn, wb_ref[...],
                                      preferred_element_type=jnp.float32)
    pad = jnp.zeros((_NP - _N, _HD), jnp.float32)
    a_ref[pl.ds(_N, _NP - _N), :] = pad
    bb_ref[pl.ds(_N, _NP - _N), :] = pad


_k1 = pl.pallas_call(
    _k1_body,
    out_shape=(jax.ShapeDtypeStruct((_NP, _HD), jnp.float32),
               jax.ShapeDtypeStruct((_NP, _HD), jnp.float32)),
)


def _make_gather(half):
    gbase0 = half * _CH

    @functools.partial(
        pl.kernel,
        out_type=jax.ShapeDtypeStruct((_EPPH, 128), jnp.float32),
        mesh=_sc_mesh,
        scratch_types=(
            pltpu.VMEM((_CPW, _CB), jnp.int32),
            pltpu.VMEM((_CPW, _CB), jnp.int32),
            pltpu.VMEM((_CB, _HD), jnp.float32),
            pltpu.VMEM((_CB, _HD), jnp.float32),
            pltpu.VMEM((_CB, _HD), jnp.float32),
            pltpu.VMEM((_CB, _HD), jnp.float32),
            pltpu.VMEM((_CPP, 128), jnp.float32),
            pltpu.VMEM((_CPP, 128), jnp.float32),
            pltpu.VMEM_SHARED((_NP, _HD), jnp.float32),
            pltpu.VMEM_SHARED((_NP, _HD), jnp.float32),
            pltpu.SemaphoreType.DMA,
            pltpu.SemaphoreType.DMA,
            pltpu.SemaphoreType.DMA,
            pltpu.SemaphoreType.DMA,
            pltpu.SemaphoreType.DMA,
            pltpu.SemaphoreType.DMA,
        ),
        compiler_params=pltpu.CompilerParams(use_tc_tiling_on_sc=False),
    )
    def _gather_sc(dst_hbm, src_hbm, a_hbm, b_hbm, p_hbm,
                   dstv, srcv, bufa0, bufb0, bufa1, bufb1, bufp0, bufp1,
                   a_sh, b_sh, sa0, sb0, sa1, sb1, sw0, sw1):
        c = lax.axis_index("c")
        s = lax.axis_index("s")
        w = s * _NC + c
        lbase = w * _CPW
        rps = _NP // _NSC
        pltpu.sync_copy(a_hbm.at[pl.ds(s * rps, rps)],
                        a_sh.at[pl.ds(s * rps, rps)])
        pltpu.sync_copy(b_hbm.at[pl.ds(s * rps, rps)],
                        b_sh.at[pl.ds(s * rps, rps)])
        pltpu.sync_copy(dst_hbm.at[pl.ds(gbase0 + lbase, _CPW)], dstv)
        pltpu.sync_copy(src_hbm.at[pl.ds(gbase0 + lbase, _CPW)], srcv)
        plsc.subcore_barrier()

        rings = ((bufa0, bufb0, bufp0, sa0, sb0, sw0),
                 (bufa1, bufb1, bufp1, sa1, sb1, sw1))

        def fire(j, ba, bb, sa, sb):
            pltpu.async_copy(a_sh.at[dstv.at[j]], ba, sa)
            pltpu.async_copy(b_sh.at[srcv.at[j]], bb, sb)

        for b in range(2):
            ba, bb, _, sa, sb, _ = rings[b]
            fire(b, ba, bb, sa, sb)

        def pair(k, carry):
            for b in range(2):
                ba, bb, bp, sa, sb, sw = rings[b]
                j = k * 2 + b
                pltpu.make_async_copy(a_sh.at[dstv.at[j]], ba, sa).wait()
                pltpu.make_async_copy(b_sh.at[srcv.at[j]], bb, sb).wait()

                @pl.when(k > 0)
                def _():
                    pltpu.make_async_copy(
                        bp, p_hbm.at[pl.ds((lbase + j - 2) * _CPP, _CPP)], sw
                    ).wait()

                @plsc.parallel_loop(0, _CPP, unroll=8)
                def rp(r):
                    for l in range(8):
                        e = r * 4 + l // 2
                        f0 = (l % 2) * 16
                        bp[r, pl.ds(l * 16, 16)] = (
                            ba[e, pl.ds(f0, 16)] + bb[e, pl.ds(f0, 16)])

                pltpu.async_copy(
                    bp, p_hbm.at[pl.ds((lbase + j) * _CPP, _CPP)], sw)

                @pl.when(k < _CPW // 2 - 1)
                def _():
                    fire(j + 2, ba, bb, sa, sb)

            return carry

        lax.fori_loop(0, _CPW // 2, pair, 0)

        for b in range(2):
            _, _, bp, _, _, sw = rings[b]
            j = _CPW - 2 + b
            pltpu.make_async_copy(
                bp, p_hbm.at[pl.ds((lbase + j) * _CPP, _CPP)], sw).wait()

    return _gather_sc


_gather0 = _make_gather(0)


_RB = 16384


def _k3_body(p_ref, w2d_ref, b1_ref, b2_ref, o_ref):
    h = jnp.maximum(p_ref[...] + b1_ref[...], 0.0)
    h = jnp.dot(h.astype(jnp.bfloat16), w2d_ref[...],
                preferred_element_type=jnp.float32) + b2_ref[...]
    o_ref[...] = jnp.maximum(h, 0.0)


_k3 = pl.pallas_call(
    _k3_body,
    grid=(_EPPH // _RB,),
    in_specs=[
        pl.BlockSpec((_RB, 128), lambda i: (i, 0)),
        pl.BlockSpec((128, 128), lambda i: (0, 0)),
        pl.BlockSpec((1, 128), lambda i: (0, 0)),
        pl.BlockSpec((1, 128), lambda i: (0, 0)),
    ],
    out_specs=pl.BlockSpec((_RB, 128), lambda i: (i, 0)),
    out_shape=jax.ShapeDtypeStruct((_EPPH, 128), jnp.float32),
)


def _make_scatter(half):
    gbase0 = half * _CH

    @functools.partial(
        pl.kernel,
        out_type=(jax.ShapeDtypeStruct((_NC, _NACC, _HD), jnp.float32),
                  jax.ShapeDtypeStruct((_NC, _NACC), jnp.float32)),
        mesh=_sc_mesh,
        scratch_types=(
            pltpu.VMEM((_CPW, _CB), jnp.int32),
            pltpu.VMEM((_CPP, 128), jnp.float32),
            pltpu.VMEM((_CPP, 128), jnp.float32),
            pltpu.VMEM((_CB, _HD), jnp.float32),
            pltpu.VMEM((_CB, _HD), jnp.float32),
            pltpu.VMEM((_CB, _HD), jnp.float32),
            pltpu.VMEM((_CB,), jnp.float32),
            pltpu.VMEM((_CB,), jnp.float32),
            pltpu.VMEM_SHARED((_NACC, _HD), jnp.float32),
            pltpu.VMEM_SHARED((_NACC,), jnp.float32),
            pltpu.SemaphoreType.DMA,
            pltpu.SemaphoreType.DMA,
            pltpu.SemaphoreType.DMA,
            pltpu.SemaphoreType.DMA,
            pltpu.SemaphoreType.DMA,
            pltpu.SemaphoreType.DMA,
        ),
        compiler_params=pltpu.CompilerParams(use_tc_tiling_on_sc=False),
    )
    def _scatter_sc(dst_hbm, h_hbm, s_out, c_out,
                    dstv, bufp0, bufp1, bufh0, bufh1, zero2d, ones1, zero1,
                    s_sh, c_sh, sr0, sr1, ss0, ss1, sc0, sc1):
        c = lax.axis_index("c")
        s = lax.axis_index("s")
        w = s * _NC + c
        lbase = w * _CPW
        rbase = s * _RPT

        rings = ((bufp0, bufh0, sr0, ss0, sc0), (bufp1, bufh1, sr1, ss1, sc1))

        for b in range(2):
            bp, _, sr, _, _ = rings[b]
            pltpu.async_copy(h_hbm.at[pl.ds((lbase + b) * _CPP, _CPP)], bp, sr)

        def fill2(i, carry):
            zero2d[i, pl.ds(0, 16)] = jnp.zeros((16,), jnp.float32)
            zero2d[i, pl.ds(16, 16)] = jnp.zeros((16,), jnp.float32)
            return carry

        lax.fori_loop(0, _CB, fill2, 0)

        def fill1(i, carry):
            ones1[pl.ds(i * 16, 16)] = jnp.ones((16,), jnp.float32)
            zero1[pl.ds(i * 16, 16)] = jnp.zeros((16,), jnp.float32)
            return carry

        lax.fori_loop(0, _CB // 16, fill1, 0)

        def zz(k, carry):
            pltpu.sync_copy(zero2d, s_sh.at[pl.ds(rbase + k * _CB, _CB)])
            pltpu.sync_copy(zero1, c_sh.at[pl.ds(rbase + k * _CB, _CB)])
            return carry

        lax.fori_loop(0, _RPT // _CB, zz, 0)
        pltpu.sync_copy(dst_hbm.at[pl.ds(gbase0 + lbase, _CPW)], dstv)
        plsc.subcore_barrier()

        def pair(k, carry):
            for b in range(2):
                bp, bh, sr, ss, sc2 = rings[b]
                j = k * 2 + b
                pltpu.make_async_copy(
                    h_hbm.at[pl.ds((lbase + j) * _CPP, _CPP)], bp, sr).wait()

                @pl.when(k > 0)
                def _():
                    pltpu.make_async_copy(
                        bh, s_sh.at[dstv.at[j - 2]], ss).wait()
                    pltpu.make_async_copy(
                        ones1, c_sh.at[dstv.at[j - 2]], sc2).wait()

                @plsc.parallel_loop(0, _CPP, unroll=8)
                def rp(r):
                    for l in range(8):
                        e = r * 4 + l // 2
                        f0 = (l % 2) * 16
                        bh[e, pl.ds(f0, 16)] = bp[r, pl.ds(l * 16, 16)]

                pltpu.async_copy(bh, s_sh.at[dstv.at[j]], ss, add=True)
                pltpu.async_copy(ones1, c_sh.at[dstv.at[j]], sc2, add=True)

                @pl.when(k < _CPW // 2 - 1)
                def _():
                    pltpu.async_copy(
                        h_hbm.at[pl.ds((lbase + j + 2) * _CPP, _CPP)], bp, sr)

            return carry

        lax.fori_loop(0, _CPW // 2, pair, 0)

        for b in range(2):
            bp, bh, sr, ss, sc2 = rings[b]
            j = _CPW - 2 + b
            pltpu.make_async_copy(bh, s_sh.at[dstv.at[j]], ss).wait()
            pltpu.make_async_copy(ones1, c_sh.at[dstv.at[j]], sc2).wait()
        plsc.subcore_barrier()

        def co(k, carry):
            pltpu.sync_copy(s_sh.at[pl.ds(rbase + k * _CB, _CB)],
                            s_out.at[c].at[pl.ds(rbase + k * _CB, _CB)])
            pltpu.sync_copy(c_sh.at[pl.ds(rbase + k * _CB, _CB)],
                            c_out.at[c].at[pl.ds(rbase + k * _CB, _CB)])
            return carry

        lax.fori_loop(0, _RPT // _CB, co, 0)

    return _scatter_sc


_scatter0 = _make_scatter(0)


def _k5_body(s0_ref, c0_ref, batch_ref, u_ref, gg_ref, gb_ref,
             w3_ref, b3_ref, wo1_ref, bo1_ref, wo2_ref, bo2_ref, wo3_ref,
             bo3_ref, o_ref):
    sv0 = s0_ref[...]
    big = (sv0[0] + sv0[1])[0:_N]
    sw = jnp.dot(big, w3_ref[...], preferred_element_type=jnp.float32)
    cv0 = c0_ref[...]
    crow = cv0[0:1, 0:_N] + cv0[1:2, 0:_N]
    recip = 1.0 / jnp.maximum(crow, 1.0)
    ind = (crow > 0.0).astype(jnp.float32)
    bt = batch_ref[...]
    gi = lax.broadcasted_iota(jnp.int32, (_G, _N), 0)
    m = (gi == bt).astype(jnp.float32)
    gs = jnp.dot(m * recip, sw, preferred_element_type=jnp.float32)
    nb = jnp.sum(m * ind, axis=1, keepdims=True)
    gcnt = jnp.sum(m, axis=1, keepdims=True)
    u2 = (gs + nb * b3_ref[...]) / jnp.maximum(gcnt, 1.0)
    uv = u_ref[...]
    mu = jnp.mean(uv, axis=0, keepdims=True)
    uc = uv - mu
    var = jnp.mean(uc * uc, axis=0, keepdims=True)
    u1 = uc * (gg_ref[...] * lax.rsqrt(var + 1e-5)) + gb_ref[...]
    uu = jnp.concatenate([u1, u2], axis=1)
    h = jnp.maximum(jnp.dot(uu, wo1_ref[...],
                            preferred_element_type=jnp.float32) + bo1_ref[...], 0.0)
    h = jnp.maximum(jnp.dot(h, wo2_ref[...],
                            preferred_element_type=jnp.float32) + bo2_ref[...], 0.0)
    o_ref[...] = jnp.dot(h, wo3_ref[...],
                         preferred_element_type=jnp.float32) + bo3_ref[...]


_k5 = pl.pallas_call(
    _k5_body,
    out_shape=jax.ShapeDtypeStruct((_G, 1), jnp.float32),
)


def kernel(x, edge_index, u, batch, bn_g, bn_b, bng_g, bng_b,
           w1, b1, w2, b2, w3, b3, wo1, bo1, wo2, bo2, wo3, bo3):
    wa = w1[:_D] - w1[_D:]
    wb = w1[_D:]
    a_t, b_t = _k1(x, bn_g.reshape(1, _D), bn_b.reshape(1, _D), wa, wb)

    pad = jnp.full((_EP - _E,), _N, jnp.int32)
    dst2 = jnp.concatenate([edge_index[1], pad]).reshape(_NCHUNK, _CB)
    src2 = jnp.concatenate([edge_index[0], pad]).reshape(_NCHUNK, _CB)

    w2d = jnp.kron(jnp.eye(4, dtype=jnp.float32), w2)
    b1t = jnp.tile(b1, 4).reshape(1, 128)
    b2t = jnp.tile(b2, 4).reshape(1, 128)

    p0 = _gather0(dst2, src2, a_t, b_t)
    h0 = _k3(p0, w2d.astype(jnp.bfloat16), b1t, b2t)
    s_p0, c_p0 = _scatter0(dst2, h0)

    return _k5(s_p0, c_p0, batch.reshape(1, _N), u,
               bng_g.reshape(1, 2), bng_b.reshape(1, 2),
               w3, b3.reshape(1, _HD),
               wo1, bo1.reshape(1, -1), wo2, bo2.reshape(1, -1),
               wo3, bo3.reshape(1, 1))


# speedup vs baseline: 1.0008x; 1.0008x over previous
import functools

import jax
import jax.numpy as jnp
from jax import lax
from jax.experimental import pallas as pl
from jax.experimental.pallas import tpu as pltpu
from jax.experimental.pallas import tpu_sc as plsc

_N = 10000
_D = 128
_HD = 32
_E = 320000
_G = 8

_NC = 2
_NSC = 16
_NW = _NC * _NSC
_CB = 128
_NCHUNK = 2560
_EP = _NCHUNK * _CB
_CH = _NCHUNK
_CPW = _CH // _NW
_CPP = _CB // 4
_EPPH = _CH * _CPP
_NP = 10016
_NACC = 10240
_RPT = _NACC // _NSC

_sc_mesh = plsc.VectorSubcoreMesh(
    core_axis_name="c", subcore_axis_name="s", num_cores=_NC, num_subcores=_NSC)


def _k1_body(x_ref, g_ref, b_ref, wa_ref, wb_ref, a_ref, bb_ref):
    x = x_ref[...]
    mu = jnp.mean(x, axis=0, keepdims=True)
    xc = x - mu
    var = jnp.mean(xc * xc, axis=0, keepdims=True)
    xn = xc * (g_ref[...] * lax.rsqrt(var + 1e-5)) + b_ref[...]
    a_ref[pl.ds(0, _N), :] = jnp.dot(xn, wa_ref[...],
                                     preferred_element_type=jnp.float32)
    bb_ref[pl.ds(0, _N), :] = jnp.dot(xn, wb_ref[...],
                                      preferred_element_type=jnp.float32)
    pad = jnp.zeros((_NP - _N, _HD), jnp.float32)
    a_ref[pl.ds(_N, _NP - _N), :] = pad
    bb_ref[pl.ds(_N, _NP - _N), :] = pad


_k1 = pl.pallas_call(
    _k1_body,
    out_shape=(jax.ShapeDtypeStruct((_NP, _HD), jnp.float32),
               jax.ShapeDtypeStruct((_NP, _HD), jnp.float32)),
)


def _make_gather(half):
    gbase0 = half * _CH

    @functools.partial(
        pl.kernel,
        out_type=jax.ShapeDtypeStruct((_EPPH, 128), jnp.float32),
        mesh=_sc_mesh,
        scratch_types=(
            pltpu.VMEM((_CPW, _CB), jnp.int32),
            pltpu.VMEM((_CPW, _CB), jnp.int32),
            pltpu.VMEM((_CB, _HD), jnp.float32),
            pltpu.VMEM((_CB, _HD), jnp.float32),
            pltpu.VMEM((_CB, _HD), jnp.float32),
            pltpu.VMEM((_CB, _HD), jnp.float32),
            pltpu.VMEM((_CPP, 128), jnp.float32),
            pltpu.VMEM((_CPP, 128), jnp.float32),
            pltpu.VMEM_SHARED((_NP, _HD), jnp.float32),
            pltpu.VMEM_SHARED((_NP, _HD), jnp.float32),
            pltpu.SemaphoreType.DMA,
            pltpu.SemaphoreType.DMA,
            pltpu.SemaphoreType.DMA,
            pltpu.SemaphoreType.DMA,
            pltpu.SemaphoreType.DMA,
            pltpu.SemaphoreType.DMA,
        ),
        compiler_params=pltpu.CompilerParams(use_tc_tiling_on_sc=False),
    )
    def _gather_sc(dst_hbm, src_hbm, a_hbm, b_hbm, p_hbm,
                   dstv, srcv, bufa0, bufb0, bufa1, bufb1, bufp0, bufp1,
                   a_sh, b_sh, sa0, sb0, sa1, sb1, sw0, sw1):
        c = lax.axis_index("c")
        s = lax.axis_index("s")
        w = s * _NC + c
        lbase = w * _CPW
        rps = _NP // _NSC
        pltpu.sync_copy(a_hbm.at[pl.ds(s * rps, rps)],
                        a_sh.at[pl.ds(s * rps, rps)])
        pltpu.sync_copy(b_hbm.at[pl.ds(s * rps, rps)],
                        b_sh.at[pl.ds(s * rps, rps)])
        pltpu.sync_copy(dst_hbm.at[pl.ds(gbase0 + lbase, _CPW)], dstv)
        pltpu.sync_copy(src_hbm.at[pl.ds(gbase0 + lbase, _CPW)], srcv)
        plsc.subcore_barrier()

        rings = ((bufa0, bufb0, bufp0, sa0, sb0, sw0),
                 (bufa1, bufb1, bufp1, sa1, sb1, sw1))

        def fire(j, ba, bb, sa, sb):
            pltpu.async_copy(a_sh.at[dstv.at[j]], ba, sa)
            pltpu.async_copy(b_sh.at[srcv.at[j]], bb, sb)

        for b in range(2):
            ba, bb, _, sa, sb, _ = rings[b]
            fire(b, ba, bb, sa, sb)

        def pair(k, carry):
            for b in range(2):
                ba, bb, bp, sa, sb, sw = rings[b]
                j = k * 2 + b
                pltpu.make_async_copy(a_sh.at[dstv.at[j]], ba, sa).wait()
                pltpu.make_async_copy(b_sh.at[srcv.at[j]], bb, sb).wait()

                @pl.when(k > 0)
                def _():
                    pltpu.make_async_copy(
                        bp, p_hbm.at[pl.ds((lbase + j - 2) * _CPP, _CPP)], sw
                    ).wait()

                @plsc.parallel_loop(0, _CPP, unroll=8)
                def rp(r):
                    for l in range(8):
                        e = r * 4 + l // 2
                        f0 = (l % 2) * 16
                        bp[r, pl.ds(l * 16, 16)] = (
                            ba[e, pl.ds(f0, 16)] + bb[e, pl.ds(f0, 16)])

                pltpu.async_copy(
                    bp, p_hbm.at[pl.ds((lbase + j) * _CPP, _CPP)], sw)

                @pl.when(k < _CPW // 2 - 1)
                def _():
                    fire(j + 2, ba, bb, sa, sb)

            return carry

        lax.fori_loop(0, _CPW // 2, pair, 0)

        for b in range(2):
            _, _, bp, _, _, sw = rings[b]
            j = _CPW - 2 + b
            pltpu.make_async_copy(
                bp, p_hbm.at[pl.ds((lbase + j) * _CPP, _CPP)], sw).wait()

    return _gather_sc


_gather0 = _make_gather(0)


_RB = 16384


def _k3_body(p_ref, w2d_ref, b1_ref, b2_ref, o_ref):
    h = jnp.maximum(p_ref[...] + b1_ref[...], 0.0)
    h = jnp.dot(h, w2d_ref[...], preferred_element_type=jnp.float32) + b2_ref[...]
    o_ref[...] = jnp.maximum(h, 0.0)


_k3 = pl.pallas_call(
    _k3_body,
    grid=(_EPPH // _RB,),
    in_specs=[
        pl.BlockSpec((_RB, 128), lambda i: (i, 0)),
        pl.BlockSpec((128, 128), lambda i: (0, 0)),
        pl.BlockSpec((1, 128), lambda i: (0, 0)),
        pl.BlockSpec((1, 128), lambda i: (0, 0)),
    ],
    out_specs=pl.BlockSpec((_RB, 128), lambda i: (i, 0)),
    out_shape=jax.ShapeDtypeStruct((_EPPH, 128), jnp.float32),
)


def _make_scatter(half):
    gbase0 = half * _CH

    @functools.partial(
        pl.kernel,
        out_type=(jax.ShapeDtypeStruct((_NC, _NACC, _HD), jnp.float32),
                  jax.ShapeDtypeStruct((_NC, _NACC), jnp.float32)),
        mesh=_sc_mesh,
        scratch_types=(
            pltpu.VMEM((_CPW, _CB), jnp.int32),
            pltpu.VMEM((_CPP, 128), jnp.float32),
            pltpu.VMEM((_CPP, 128), jnp.float32),
            pltpu.VMEM((_CB, _HD), jnp.float32),
            pltpu.VMEM((_CB, _HD), jnp.float32),
            pltpu.VMEM((_CB, _HD), jnp.float32),
            pltpu.VMEM((_CB,), jnp.float32),
            pltpu.VMEM((_CB,), jnp.float32),
            pltpu.VMEM_SHARED((_NACC, _HD), jnp.float32),
            pltpu.VMEM_SHARED((_NACC,), jnp.float32),
            pltpu.SemaphoreType.DMA,
            pltpu.SemaphoreType.DMA,
            pltpu.SemaphoreType.DMA,
            pltpu.SemaphoreType.DMA,
            pltpu.SemaphoreType.DMA,
            pltpu.SemaphoreType.DMA,
        ),
        compiler_params=pltpu.CompilerParams(use_tc_tiling_on_sc=False),
    )
    def _scatter_sc(dst_hbm, h_hbm, s_out, c_out,
                    dstv, bufp0, bufp1, bufh0, bufh1, zero2d, ones1, zero1,
                    s_sh, c_sh, sr0, sr1, ss0, ss1, sc0, sc1):
        c = lax.axis_index("c")
        s = lax.axis_index("s")
        w = s * _NC + c
        lbase = w * _CPW
        rbase = s * _RPT

        rings = ((bufp0, bufh0, sr0, ss0, sc0), (bufp1, bufh1, sr1, ss1, sc1))

        for b in range(2):
            bp, _, sr, _, _ = rings[b]
            pltpu.async_copy(h_hbm.at[pl.ds((lbase + b) * _CPP, _CPP)], bp, sr)

        def fill2(i, carry):
            zero2d[i, pl.ds(0, 16)] = jnp.zeros((16,), jnp.float32)
            zero2d[i, pl.ds(16, 16)] = jnp.zeros((16,), jnp.float32)
            return carry

        lax.fori_loop(0, _CB, fill2, 0)

        def fill1(i, carry):
            ones1[pl.ds(i * 16, 16)] = jnp.ones((16,), jnp.float32)
            zero1[pl.ds(i * 16, 16)] = jnp.zeros((16,), jnp.float32)
            return carry

        lax.fori_loop(0, _CB // 16, fill1, 0)

        def zz(k, carry):
            pltpu.sync_copy(zero2d, s_sh.at[pl.ds(rbase + k * _CB, _CB)])
            pltpu.sync_copy(zero1, c_sh.at[pl.ds(rbase + k * _CB, _CB)])
            return carry

        lax.fori_loop(0, _RPT // _CB, zz, 0)
        pltpu.sync_copy(dst_hbm.at[pl.ds(gbase0 + lbase, _CPW)], dstv)
        plsc.subcore_barrier()

        def pair(k, carry):
            for b in range(2):
                bp, bh, sr, ss, sc2 = rings[b]
                j = k * 2 + b
                pltpu.make_async_copy(
                    h_hbm.at[pl.ds((lbase + j) * _CPP, _CPP)], bp, sr).wait()

                @pl.when(k > 0)
                def _():
                    pltpu.make_async_copy(
                        bh, s_sh.at[dstv.at[j - 2]], ss).wait()
                    pltpu.make_async_copy(
                        ones1, c_sh.at[dstv.at[j - 2]], sc2).wait()

                @plsc.parallel_loop(0, _CPP, unroll=8)
                def rp(r):
                    for l in range(8):
                        e = r * 4 + l // 2
                        f0 = (l % 2) * 16
                        bh[e, pl.ds(f0, 16)] = bp[r, pl.ds(l * 16, 16)]

                pltpu.async_copy(bh, s_sh.at[dstv.at[j]], ss, add=True)
                pltpu.async_copy(ones1, c_sh.at[dstv.at[j]], sc2, add=True)

                @pl.when(k < _CPW // 2 - 1)
                def _():
                    pltpu.async_copy(
                        h_hbm.at[pl.ds((lbase + j + 2) * _CPP, _CPP)], bp, sr)

            return carry

        lax.fori_loop(0, _CPW // 2, pair, 0)

        for b in range(2):
            bp, bh, sr, ss, sc2 = rings[b]
            j = _CPW - 2 + b
            pltpu.make_async_copy(bh, s_sh.at[dstv.at[j]], ss).wait()
            pltpu.make_async_copy(ones1, c_sh.at[dstv.at[j]], sc2).wait()
        plsc.subcore_barrier()

        def co(k, carry):
            pltpu.sync_copy(s_sh.at[pl.ds(rbase + k * _CB, _CB)],
                            s_out.at[c].at[pl.ds(rbase + k * _CB, _CB)])
            pltpu.sync_copy(c_sh.at[pl.ds(rbase + k * _CB, _CB)],
                            c_out.at[c].at[pl.ds(rbase + k * _CB, _CB)])
            return carry

        lax.fori_loop(0, _RPT // _CB, co, 0)

    return _scatter_sc


_scatter0 = _make_scatter(0)


def _k5_body(s0_ref, c0_ref, batch_ref, u_ref, gg_ref, gb_ref,
             w3_ref, b3_ref, wo1_ref, bo1_ref, wo2_ref, bo2_ref, wo3_ref,
             bo3_ref, o_ref):
    sv0 = s0_ref[...]
    big = (sv0[0] + sv0[1])[0:_N]
    sw = jnp.dot(big, w3_ref[...], preferred_element_type=jnp.float32)
    cv0 = c0_ref[...]
    crow = cv0[0:1, 0:_N] + cv0[1:2, 0:_N]
    recip = 1.0 / jnp.maximum(crow, 1.0)
    ind = (crow > 0.0).astype(jnp.float32)
    bt = batch_ref[...]
    gi = lax.broadcasted_iota(jnp.int32, (_G, _N), 0)
    m = (gi == bt).astype(jnp.float32)
    gs = jnp.dot(m * recip, sw, preferred_element_type=jnp.float32)
    nb = jnp.sum(m * ind, axis=1, keepdims=True)
    gcnt = jnp.sum(m, axis=1, keepdims=True)
    u2 = (gs + nb * b3_ref[...]) / jnp.maximum(gcnt, 1.0)
    uv = u_ref[...]
    mu = jnp.mean(uv, axis=0, keepdims=True)
    uc = uv - mu
    var = jnp.mean(uc * uc, axis=0, keepdims=True)
    u1 = uc * (gg_ref[...] * lax.rsqrt(var + 1e-5)) + gb_ref[...]
    uu = jnp.concatenate([u1, u2], axis=1)
    h = jnp.maximum(jnp.dot(uu, wo1_ref[...],
                            preferred_element_type=jnp.float32) + bo1_ref[...], 0.0)
    h = jnp.maximum(jnp.dot(h, wo2_ref[...],
                            preferred_element_type=jnp.float32) + bo2_ref[...], 0.0)
    o_ref[...] = jnp.dot(h, wo3_ref[...],
                         preferred_element_type=jnp.float32) + bo3_ref[...]


_k5 = pl.pallas_call(
    _k5_body,
    out_shape=jax.ShapeDtypeStruct((_G, 1), jnp.float32),
)


def kernel(x, edge_index, u, batch, bn_g, bn_b, bng_g, bng_b,
           w1, b1, w2, b2, w3, b3, wo1, bo1, wo2, bo2, wo3, bo3):
    wa = w1[:_D] - w1[_D:]
    wb = w1[_D:]
    a_t, b_t = _k1(x, bn_g.reshape(1, _D), bn_b.reshape(1, _D), wa, wb)

    pad = jnp.full((_EP - _E,), _N, jnp.int32)
    dst2 = jnp.concatenate([edge_index[1], pad]).reshape(_NCHUNK, _CB)
    src2 = jnp.concatenate([edge_index[0], pad]).reshape(_NCHUNK, _CB)

    w2d = jnp.kron(jnp.eye(4, dtype=jnp.float32), w2)
    b1t = jnp.tile(b1, 4).reshape(1, 128)
    b2t = jnp.tile(b2, 4).reshape(1, 128)

    p0 = _gather0(dst2, src2, a_t, b_t)
    h0 = _k3(p0, w2d, b1t, b2t)
    s_p0, c_p0 = _scatter0(dst2, h0)

    return _k5(s_p0, c_p0, batch.reshape(1, _N), u,
               bng_g.reshape(1, 2), bng_b.reshape(1, 2),
               w3, b3.reshape(1, _HD),
               wo1, bo1.reshape(1, -1), wo2, bo2.reshape(1, -1),
               wo3, bo3.reshape(1, 1))
